# baseline (speedup 1.0000x reference)
import jax
import jax.numpy as jnp
from jax.experimental import pallas as pl
from jax.experimental.pallas import tpu as pltpu

TOKEN_BLOCK = 512


def _router_kernel(x_ref, w_ref, noise_ref, out_ref):
    logits_t = jax.lax.dot_general(
        w_ref[...],
        x_ref[...],
        dimension_numbers=(((1,), (1,)), ((), ())),
        preferred_element_type=jnp.float32,
    )
    out_ref[...] = logits_t + noise_ref[...]


def kernel(x, W, noise):
    tokens, d_model = x.shape
    n_experts = W.shape[0]
    noise_t = noise.T
    grid = (tokens // TOKEN_BLOCK,)
    out_t = pl.pallas_call(
        _router_kernel,
        grid=grid,
        in_specs=[
            pl.BlockSpec((TOKEN_BLOCK, d_model), lambda i: (i, 0)),
            pl.BlockSpec((n_experts, d_model), lambda i: (0, 0)),
            pl.BlockSpec((n_experts, TOKEN_BLOCK), lambda i: (0, i)),
        ],
        out_specs=pl.BlockSpec((n_experts, TOKEN_BLOCK), lambda i: (0, i)),
        out_shape=jax.ShapeDtypeStruct((n_experts, tokens), jnp.float32),
        compiler_params=pltpu.CompilerParams(
            dimension_semantics=("arbitrary",),
        ),
    )(x, W, noise_t)
    return out_t.T


# speedup vs baseline: 1.0067x; 1.0067x over previous
import jax
import jax.numpy as jnp
from jax.experimental import pallas as pl
from jax.experimental.pallas import tpu as pltpu

CHUNK = 512
BUFS = 4


def _outer(x_hbm, w_hbm, noise_hbm, out_hbm):
    n_experts, d_model = w_hbm.shape
    tokens = x_hbm.shape[0]

    def body(x_blk, w_blk, n_blk, o_blk):
        logits_t = jax.lax.dot_general(
            w_blk[...],
            x_blk[...],
            dimension_numbers=(((1,), (1,)), ((), ())),
            preferred_element_type=jnp.float32,
        )
        o_blk[...] = logits_t + n_blk[...]

    pltpu.emit_pipeline(
        body,
        grid=(tokens // CHUNK,),
        in_specs=[
            pl.BlockSpec((CHUNK, d_model), lambda i: (i, 0),
                         pipeline_mode=pl.Buffered(buffer_count=BUFS)),
            pl.BlockSpec((n_experts, d_model), lambda i: (0, 0)),
            pl.BlockSpec((n_experts, CHUNK), lambda i: (0, i)),
        ],
        out_specs=[pl.BlockSpec((n_experts, CHUNK), lambda i: (0, i))],
    )(x_hbm, w_hbm, noise_hbm, out_hbm)


def kernel(x, W, noise):
    tokens, d_model = x.shape
    n_experts = W.shape[0]
    noise_t = noise.T
    out_t = pl.pallas_call(
        _outer,
        in_specs=[
            pl.BlockSpec(memory_space=pltpu.MemorySpace.HBM),
            pl.BlockSpec(memory_space=pltpu.MemorySpace.HBM),
            pl.BlockSpec(memory_space=pltpu.MemorySpace.HBM),
        ],
        out_specs=pl.BlockSpec(memory_space=pltpu.MemorySpace.HBM),
        out_shape=jax.ShapeDtypeStruct((n_experts, tokens), jnp.float32),
    )(x, W, noise_t)
    return out_t.T


# speedup vs baseline: 1.1506x; 1.1430x over previous
import jax
import jax.numpy as jnp
from jax.experimental import pallas as pl
from jax.experimental.pallas import tpu as pltpu

TOKEN_BLOCK = 1024


def _router_kernel(x_ref, w_ref, noise_ref, out_ref):
    logits_t = jax.lax.dot_general(
        w_ref[...].astype(jnp.bfloat16),
        x_ref[...].astype(jnp.bfloat16),
        dimension_numbers=(((1,), (1,)), ((), ())),
        preferred_element_type=jnp.float32,
    )
    out_ref[...] = logits_t + noise_ref[...]


def kernel(x, W, noise):
    tokens, d_model = x.shape
    n_experts = W.shape[0]
    noise_t = noise.T
    grid = (tokens // TOKEN_BLOCK,)
    out_t = pl.pallas_call(
        _router_kernel,
        grid=grid,
        in_specs=[
            pl.BlockSpec((TOKEN_BLOCK, d_model), lambda i: (i, 0)),
            pl.BlockSpec((n_experts, d_model), lambda i: (0, 0)),
            pl.BlockSpec((n_experts, TOKEN_BLOCK), lambda i: (0, i)),
        ],
        out_specs=pl.BlockSpec((n_experts, TOKEN_BLOCK), lambda i: (0, i)),
        out_shape=jax.ShapeDtypeStruct((n_experts, tokens), jnp.float32),
        compiler_params=pltpu.CompilerParams(
            dimension_semantics=("arbitrary",),
        ),
    )(x, W, noise_t)
    return out_t.T


# speedup vs baseline: 1.2058x; 1.0479x over previous
import jax
import jax.numpy as jnp
from jax.experimental import pallas as pl
from jax.experimental.pallas import tpu as pltpu

TOKEN_BLOCK = 1024


def _router_kernel(x_ref, w_ref, noise_ref, out_ref):
    i = pl.program_id(0)
    logits_t = jax.lax.dot_general(
        w_ref[...],
        x_ref[...],
        dimension_numbers=(((1,), (1,)), ((), ())),
        preferred_element_type=jnp.float32,
    )
    cols = pl.ds(i * TOKEN_BLOCK, TOKEN_BLOCK)
    out_ref[...] = logits_t + noise_ref[:, cols]


def kernel(x, W, noise):
    tokens, d_model = x.shape
    n_experts = W.shape[0]
    noise_t = noise.T
    grid = (tokens // TOKEN_BLOCK,)
    out_t = pl.pallas_call(
        _router_kernel,
        grid=grid,
        in_specs=[
            pl.BlockSpec((TOKEN_BLOCK, d_model), lambda i: (i, 0)),
            pl.BlockSpec((n_experts, d_model), lambda i: (0, 0)),
            pl.BlockSpec((n_experts, tokens), lambda i: (0, 0)),
        ],
        out_specs=pl.BlockSpec((n_experts, TOKEN_BLOCK), lambda i: (0, i)),
        out_shape=jax.ShapeDtypeStruct((n_experts, tokens), jnp.float32),
        compiler_params=pltpu.CompilerParams(
            dimension_semantics=("arbitrary",),
        ),
    )(x, W, noise_t)
    return out_t.T
